# baseline (speedup 1.0000x reference)
import functools

import jax
import jax.numpy as jnp
from jax import lax
from jax.experimental import pallas as pl
from jax.experimental.pallas import tpu as pltpu
from jax.experimental.pallas import tpu_sc as plsc

N_NODES = 10000
N_EDGES = 320000
D = 128

NC = 2
NS = 16
NW = NC * NS
PER_W = N_EDGES // NW
CHUNK = 80
NCH = PER_W // CHUNK


def _mm_body(x_ref, w_ref, b_ref, y_ref):
    y_ref[...] = lax.dot_general(
        x_ref[...], w_ref[...], (((1,), (1,)), ((), ())),
        preferred_element_type=jnp.float32,
    ) + b_ref[0:1, :]


def _transform(x, W, b):
    blk = 1000
    return pl.pallas_call(
        _mm_body,
        grid=(N_NODES // blk,),
        in_specs=[
            pl.BlockSpec((blk, D), lambda i: (i, 0)),
            pl.BlockSpec((D, D), lambda i: (0, 0)),
            pl.BlockSpec((8, D), lambda i: (0, 0)),
        ],
        out_specs=pl.BlockSpec((blk, D), lambda i: (i, 0)),
        out_shape=jax.ShapeDtypeStruct((N_NODES, D), jnp.float32),
    )(x, W, jnp.broadcast_to(b.reshape(1, D), (8, D)))


NBUF = 5
PREF = 2


def _gather_body(y_hbm, idx_hbm, out_hbm, idx_v, rows, gsems, osems):
    wid = lax.axis_index("s") * NC + lax.axis_index("c")
    base = wid * PER_W
    pltpu.sync_copy(idx_hbm.at[1, wid], idx_v)

    for j in range(PREF):
        pltpu.async_copy(y_hbm.at[idx_v.at[j]], rows[j], gsems[j])

    @pl.loop(0, NCH, step=NBUF)
    def _group(c):
        for j in range(NBUF):
            ch = c + j
            pltpu.make_async_copy(y_hbm.at[idx_v.at[ch]], rows[j], gsems[j]).wait()
            pltpu.async_copy(
                rows[j], out_hbm.at[pl.ds(base + ch * CHUNK, CHUNK)], osems[j])
            k = (j + PREF) % NBUF

            @pl.when(ch + PREF < NCH)
            def _():
                @pl.when(ch + PREF >= NBUF)
                def _():
                    pltpu.make_async_copy(
                        rows[k],
                        out_hbm.at[pl.ds(base + (ch + PREF - NBUF) * CHUNK, CHUNK)],
                        osems[k],
                    ).wait()
                pltpu.async_copy(y_hbm.at[idx_v.at[ch + PREF]], rows[k], gsems[k])

    for j in range(NBUF):
        ch = NCH - NBUF + j
        pltpu.make_async_copy(
            rows[j], out_hbm.at[pl.ds(base + ch * CHUNK, CHUNK)], osems[j]
        ).wait()


_gather = functools.partial(
    pl.kernel,
    out_type=jax.ShapeDtypeStruct((N_EDGES, D), jnp.float32),
    mesh=plsc.VectorSubcoreMesh(core_axis_name="c", subcore_axis_name="s"),
    scratch_types=[
        pltpu.VMEM((NCH, CHUNK), jnp.int32),
        [pltpu.VMEM((CHUNK, D), jnp.float32) for _ in range(NBUF)],
        [pltpu.SemaphoreType.DMA for _ in range(NBUF)],
        [pltpu.SemaphoreType.DMA for _ in range(NBUF)],
    ],
)(_gather_body)


@jax.jit
def kernel(x, edge_index, W, b):
    y = _transform(x, W, b)
    ei = edge_index.astype(jnp.int32).reshape(2, NW, NCH, CHUNK)
    return _gather(y, ei)


# speedup vs baseline: 1.0452x; 1.0452x over previous
import functools

import jax
import jax.numpy as jnp
from jax import lax
from jax.experimental import pallas as pl
from jax.experimental.pallas import tpu as pltpu
from jax.experimental.pallas import tpu_sc as plsc

N_NODES = 10000
N_EDGES = 320000
D = 128

NC = 2
NS = 16
NW = NC * NS
PER_W = N_EDGES // NW
CHUNK = 80
NCH = PER_W // CHUNK


def _mm_body(x_ref, w_ref, b_ref, y_ref):
    y_ref[...] = lax.dot_general(
        x_ref[...], w_ref[...], (((1,), (1,)), ((), ())),
        preferred_element_type=jnp.float32,
    ) + b_ref[0:1, :]


def _transform(x, W, b):
    blk = 1000
    return pl.pallas_call(
        _mm_body,
        grid=(N_NODES // blk,),
        in_specs=[
            pl.BlockSpec((blk, D), lambda i: (i, 0)),
            pl.BlockSpec((D, D), lambda i: (0, 0)),
            pl.BlockSpec((8, D), lambda i: (0, 0)),
        ],
        out_specs=pl.BlockSpec((blk, D), lambda i: (i, 0)),
        out_shape=jax.ShapeDtypeStruct((N_NODES, D), jnp.float32),
    )(x, W, jnp.broadcast_to(b.reshape(1, D), (8, D)))


NBUF = 5
PREF = 4


def _gather_body(y_hbm, idx_hbm, out_hbm, idx_v, rows, gsems, osems):
    wid = lax.axis_index("s") * NC + lax.axis_index("c")
    base = wid * PER_W
    pltpu.sync_copy(idx_hbm.at[1, wid], idx_v)

    for j in range(PREF):
        pltpu.async_copy(y_hbm.at[idx_v.at[j]], rows[j], gsems[j])

    @pl.loop(0, NCH, step=NBUF)
    def _group(c):
        for j in range(NBUF):
            ch = c + j
            pltpu.make_async_copy(y_hbm.at[idx_v.at[ch]], rows[j], gsems[j]).wait()
            pltpu.async_copy(
                rows[j], out_hbm.at[pl.ds(base + ch * CHUNK, CHUNK)], osems[j])
            k = (j + PREF) % NBUF

            @pl.when(ch + PREF < NCH)
            def _():
                @pl.when(ch + PREF >= NBUF)
                def _():
                    pltpu.make_async_copy(
                        rows[k],
                        out_hbm.at[pl.ds(base + (ch + PREF - NBUF) * CHUNK, CHUNK)],
                        osems[k],
                    ).wait()
                pltpu.async_copy(y_hbm.at[idx_v.at[ch + PREF]], rows[k], gsems[k])

    for j in range(NBUF):
        ch = NCH - NBUF + j
        pltpu.make_async_copy(
            rows[j], out_hbm.at[pl.ds(base + ch * CHUNK, CHUNK)], osems[j]
        ).wait()


_gather = functools.partial(
    pl.kernel,
    out_type=jax.ShapeDtypeStruct((N_EDGES, D), jnp.float32),
    mesh=plsc.VectorSubcoreMesh(core_axis_name="c", subcore_axis_name="s"),
    scratch_types=[
        pltpu.VMEM((NCH, CHUNK), jnp.int32),
        [pltpu.VMEM((CHUNK, D), jnp.float32) for _ in range(NBUF)],
        [pltpu.SemaphoreType.DMA for _ in range(NBUF)],
        [pltpu.SemaphoreType.DMA for _ in range(NBUF)],
    ],
)(_gather_body)


@jax.jit
def kernel(x, edge_index, W, b):
    y = _transform(x, W, b)
    ei = edge_index.astype(jnp.int32).reshape(2, NW, NCH, CHUNK)
    return _gather(y, ei)
